# baseline (speedup 1.0000x reference)
import jax
import jax.numpy as jnp
from jax.experimental import pallas as pl
from jax.experimental.pallas import tpu as pltpu

_BM = 512
_SUB = 4
_SM = _BM // _SUB


def _matmul_body(x_ref, w_ref, b_ref, o_ref):
    j = pl.program_id(1)
    acc = jax.lax.dot_general(
        x_ref[pl.ds(j * _SM, _SM), :],
        w_ref[...],
        dimension_numbers=(((1,), (1,)), ((), ())),
        preferred_element_type=jnp.float32,
    )
    o_ref[...] = acc + b_ref[...]


@jax.jit
def kernel(input, weight, bias):
    m, k = input.shape
    n = weight.shape[0]
    grid = (m // _BM, _SUB)
    return pl.pallas_call(
        _matmul_body,
        grid=grid,
        in_specs=[
            pl.BlockSpec(
                (_BM, k),
                lambda i, j: (i, 0),
                pipeline_mode=pl.Buffered(
                    buffer_count=2, revisit=pl.RevisitMode.IMMEDIATE
                ),
            ),
            pl.BlockSpec((n, k), lambda i, j: (0, 0)),
            pl.BlockSpec((1, n), lambda i, j: (0, 0)),
        ],
        out_specs=pl.BlockSpec((_SM, n), lambda i, j: (i * _SUB + j, 0)),
        out_shape=jax.ShapeDtypeStruct((m, n), jnp.float32),
        compiler_params=pltpu.CompilerParams(
            dimension_semantics=("arbitrary", "arbitrary"),
        ),
    )(input, weight, bias.reshape(1, n))


# speedup vs baseline: 1.1402x; 1.1402x over previous
import jax
import jax.numpy as jnp
from jax.experimental import pallas as pl
from jax.experimental.pallas import tpu as pltpu

_BM = 512
_SM = 256


def _matmul_body(x_ref, w_ref, b_ref, o_ref):
    j = pl.program_id(0) % 2
    acc = jax.lax.dot_general(
        x_ref[pl.ds(j * _SM, _SM), :],
        w_ref[...],
        dimension_numbers=(((1,), (1,)), ((), ())),
        preferred_element_type=jnp.float32,
    )
    o_ref[...] = acc + b_ref[...]


@jax.jit
def kernel(input, weight, bias):
    m, k = input.shape
    n = weight.shape[0]
    grid = (m // _SM,)
    return pl.pallas_call(
        _matmul_body,
        grid=grid,
        in_specs=[
            pl.BlockSpec(
                (_BM, k),
                lambda i: (i // 2, 0),
                pipeline_mode=pl.Buffered(
                    buffer_count=2, revisit=pl.RevisitMode.IMMEDIATE
                ),
            ),
            pl.BlockSpec((n, k), lambda i: (0, 0)),
            pl.BlockSpec((1, n), lambda i: (0, 0)),
        ],
        out_specs=pl.BlockSpec((_SM, n), lambda i: (i, 0)),
        out_shape=jax.ShapeDtypeStruct((m, n), jnp.float32),
        compiler_params=pltpu.CompilerParams(
            dimension_semantics=("arbitrary",),
        ),
    )(input, weight, bias.reshape(1, n))


# speedup vs baseline: 1.6901x; 1.4822x over previous
import jax
import jax.numpy as jnp
from jax.experimental import pallas as pl
from jax.experimental.pallas import tpu as pltpu

_BM = 512


def _matmul_body(x_ref, w_ref, b_ref, o_ref):
    acc = jax.lax.dot_general(
        x_ref[...],
        w_ref[...],
        dimension_numbers=(((1,), (1,)), ((), ())),
        preferred_element_type=jnp.float32,
    )
    o_ref[...] = acc + b_ref[...]


@jax.jit
def kernel(input, weight, bias):
    m, k = input.shape
    n = weight.shape[0]
    grid = (m // _BM,)
    return pl.pallas_call(
        _matmul_body,
        grid=grid,
        in_specs=[
            pl.BlockSpec((_BM, k), lambda i: (i, 0)),
            pl.BlockSpec((n, k), lambda i: (0, 0)),
            pl.BlockSpec((1, n), lambda i: (0, 0)),
        ],
        out_specs=pl.BlockSpec((_BM, n), lambda i: (i, 0)),
        out_shape=jax.ShapeDtypeStruct((m, n), jnp.float32),
        compiler_params=pltpu.CompilerParams(
            dimension_semantics=("parallel",),
        ),
    )(input, weight, bias.reshape(1, n))
